# Initial kernel scaffold; baseline (speedup 1.0000x reference)
import jax
import jax.numpy as jnp
from jax.experimental import pallas as pl


def kernel(token_ids, embeddings):
    raise NotImplementedError("write your pallas kernel here")


# speedup vs baseline: 1.8344x; 1.8344x over previous
import functools

import jax
import jax.numpy as jnp
from jax import lax
from jax.experimental import pallas as pl
from jax.experimental.pallas import tpu as pltpu
from jax.experimental.pallas import tpu_sc as plsc

NUM_CORES = 2
NUM_SUBCORES = 16
NUM_WORKERS = NUM_CORES * NUM_SUBCORES

CHUNK = 1024
STREAM = 128


@functools.partial(jax.jit, static_argnums=(2, 3))
def _sc_gather(flat_ids, table, b_total, d):
    b_per_w = b_total // NUM_WORKERS
    n_chunks = b_per_w // CHUNK
    mesh = plsc.VectorSubcoreMesh(core_axis_name="c", subcore_axis_name="s")

    @functools.partial(
        pl.kernel,
        mesh=mesh,
        out_type=jax.ShapeDtypeStruct((b_total, d), jnp.float32),
        scratch_types=[
            pltpu.VMEM((CHUNK,), jnp.int32),
            pltpu.VMEM((CHUNK, d), jnp.float32),
            pltpu.SemaphoreType.DMA,
        ],
        compiler_params=pltpu.CompilerParams(use_tc_tiling_on_sc=False),
    )
    def k(ids_hbm, table_hbm, out_hbm, idx_v, rows_v, sem):
        wid = lax.axis_index("s") * NUM_CORES + lax.axis_index("c")
        base = wid * b_per_w

        def body(i, carry):
            off = base + i * CHUNK
            pltpu.sync_copy(ids_hbm.at[pl.ds(off, CHUNK)], idx_v)
            for j in range(CHUNK // STREAM):
                pltpu.async_copy(
                    table_hbm.at[idx_v.at[pl.ds(j * STREAM, STREAM)]],
                    rows_v.at[pl.ds(j * STREAM, STREAM)],
                    sem,
                )
            for j in range(CHUNK // STREAM):
                pltpu.make_async_copy(
                    table_hbm.at[idx_v.at[pl.ds(j * STREAM, STREAM)]],
                    rows_v.at[pl.ds(j * STREAM, STREAM)],
                    sem,
                ).wait()
            pltpu.sync_copy(rows_v, out_hbm.at[pl.ds(off, CHUNK)])
            return carry

        lax.fori_loop(0, n_chunks, body, 0)

    return k(flat_ids, table)


def kernel(token_ids, embeddings):
    b_total = token_ids.shape[0] * token_ids.shape[1]
    d = embeddings.shape[1]
    flat = token_ids.reshape(-1).astype(jnp.int32)
    out = _sc_gather(flat, embeddings, b_total, d)
    return out.reshape(token_ids.shape + (d,))


# speedup vs baseline: 1.8570x; 1.0123x over previous
import functools

import jax
import jax.numpy as jnp
from jax import lax
from jax.experimental import pallas as pl
from jax.experimental.pallas import tpu as pltpu
from jax.experimental.pallas import tpu_sc as plsc

NUM_CORES = 2
NUM_SUBCORES = 16
NUM_WORKERS = NUM_CORES * NUM_SUBCORES

CHUNK = 640
STREAM = 128


@functools.partial(jax.jit, static_argnums=(2, 3))
def _sc_gather(flat_ids, table, b_total, d):
    b_per_w = b_total // NUM_WORKERS
    n_chunks = b_per_w // CHUNK
    n_pairs = n_chunks // 2
    n_streams = CHUNK // STREAM
    mesh = plsc.VectorSubcoreMesh(core_axis_name="c", subcore_axis_name="s")

    @functools.partial(
        pl.kernel,
        mesh=mesh,
        out_type=jax.ShapeDtypeStruct((b_total, d), jnp.float32),
        scratch_types=[
            pltpu.VMEM((b_per_w // STREAM, STREAM), jnp.int32),
            pltpu.VMEM((CHUNK, d), jnp.float32),
            pltpu.VMEM((CHUNK, d), jnp.float32),
            pltpu.SemaphoreType.DMA,
            pltpu.SemaphoreType.DMA,
            pltpu.SemaphoreType.DMA,
            pltpu.SemaphoreType.DMA,
        ],
        compiler_params=pltpu.CompilerParams(use_tc_tiling_on_sc=False),
    )
    def k(ids_hbm, table_hbm, out_hbm, ids_v, rows0, rows1, g0, g1, o0, o1):
        wid = lax.axis_index("s") * NUM_CORES + lax.axis_index("c")
        base = wid * b_per_w
        rows_per_w = b_per_w // STREAM
        pltpu.sync_copy(ids_hbm.at[pl.ds(wid * rows_per_w, rows_per_w)], ids_v)

        def fire(slot, ch, gsem):
            for j in range(n_streams):
                pltpu.async_copy(
                    table_hbm.at[ids_v.at[ch * n_streams + j]],
                    slot.at[pl.ds(j * STREAM, STREAM)],
                    gsem,
                )

        def drain(slot, ch, gsem):
            for j in range(n_streams):
                pltpu.make_async_copy(
                    table_hbm.at[ids_v.at[ch * n_streams + j]],
                    slot.at[pl.ds(j * STREAM, STREAM)],
                    gsem,
                ).wait()

        def store(slot, ch, osem):
            pltpu.async_copy(
                slot, out_hbm.at[pl.ds(base + ch * CHUNK, CHUNK)], osem
            )

        def wait_store(slot, ch, osem):
            pltpu.make_async_copy(
                slot, out_hbm.at[pl.ds(base + ch * CHUNK, CHUNK)], osem
            ).wait()

        def body(i, carry):
            c0 = 2 * i
            c1 = 2 * i + 1

            @pl.when(i > 0)
            def _():
                wait_store(rows0, c0 - 2, o0)

            fire(rows0, c0, g0)

            @pl.when(i > 0)
            def _():
                wait_store(rows1, c1 - 2, o1)

            fire(rows1, c1, g1)
            drain(rows0, c0, g0)
            store(rows0, c0, o0)
            drain(rows1, c1, g1)
            store(rows1, c1, o1)
            return carry

        lax.fori_loop(0, n_pairs, body, 0)
        wait_store(rows0, n_chunks - 2, o0)
        wait_store(rows1, n_chunks - 1, o1)

    return k(flat_ids, table)


def kernel(token_ids, embeddings):
    b_total = token_ids.shape[0] * token_ids.shape[1]
    d = embeddings.shape[1]
    flat = token_ids.reshape(b_total // STREAM, STREAM).astype(jnp.int32)
    out = _sc_gather(flat, embeddings, b_total, d)
    return out.reshape(token_ids.shape + (d,))
